# baseline (speedup 1.0000x reference)
import functools

import jax
import jax.numpy as jnp
from jax import lax
from jax.experimental import pallas as pl
from jax.experimental.pallas import tpu as pltpu
from jax.experimental.pallas import tpu_sc as plsc

NUM_WORKERS = 32
BLK = 512
SUB = 512
NSUB = BLK // SUB


@functools.lru_cache(maxsize=None)
def _build_gather(n: int, v: int, d: int):
    b_per_w = n // NUM_WORKERS
    n_blk = b_per_w // BLK
    assert b_per_w * NUM_WORKERS == n and n_blk * BLK == b_per_w
    assert n_blk % 2 == 0

    mesh = plsc.VectorSubcoreMesh(core_axis_name="c", subcore_axis_name="s")

    @functools.partial(
        pl.kernel,
        mesh=mesh,
        out_type=jax.ShapeDtypeStruct((n, d), jnp.float32),
        scratch_types=[
            pltpu.VMEM((b_per_w,), jnp.int32),
            pltpu.VMEM((2 * BLK, d), jnp.float32),
            pltpu.SemaphoreType.DMA,
            pltpu.SemaphoreType.DMA,
            pltpu.SemaphoreType.DMA,
            pltpu.SemaphoreType.DMA,
        ],
        compiler_params=pltpu.CompilerParams(use_tc_tiling_on_sc=False),
    )
    def gather_kernel(table_hbm, idx_hbm, out_hbm, idx_v, rows_v, sem_ga,
                      sem_gb, sem_wa, sem_wb):
        wid = lax.axis_index("s") * 2 + lax.axis_index("c")
        base = wid * b_per_w
        g_sems = (sem_ga, sem_gb)
        w_sems = (sem_wa, sem_wb)

        def fire_gathers(blk, half):
            for j in range(NSUB):
                pltpu.async_copy(
                    table_hbm.at[idx_v.at[pl.ds(blk * BLK + j * SUB, SUB)]],
                    rows_v.at[pl.ds(half * BLK + j * SUB, SUB)],
                    g_sems[half],
                )

        def drain_gathers(half):
            pltpu.make_async_copy(
                table_hbm.at[pl.ds(0, BLK)],
                rows_v.at[pl.ds(half * BLK, BLK)],
                g_sems[half],
            ).wait()

        def fire_write(blk, half):
            pltpu.async_copy(
                rows_v.at[pl.ds(half * BLK, BLK)],
                out_hbm.at[pl.ds(base + blk * BLK, BLK)],
                w_sems[half],
            )

        def drain_write(half):
            pltpu.make_async_copy(
                rows_v.at[pl.ds(half * BLK, BLK)],
                out_hbm.at[pl.ds(base, BLK)],
                w_sems[half],
            ).wait()

        pltpu.sync_copy(idx_hbm.at[pl.ds(base, b_per_w)], idx_v)
        fire_gathers(0, 0)
        fire_gathers(1, 1)

        def body(g, carry):
            b = 2 * g
            drain_gathers(0)
            fire_write(b, 0)
            drain_gathers(1)
            fire_write(b + 1, 1)
            drain_write(0)
            fire_gathers(b + 2, 0)
            drain_write(1)
            fire_gathers(b + 3, 1)
            return carry

        lax.fori_loop(0, n_blk // 2 - 1, body, 0)

        drain_gathers(0)
        fire_write(n_blk - 2, 0)
        drain_gathers(1)
        fire_write(n_blk - 1, 1)
        drain_write(0)
        drain_write(1)

    return gather_kernel


def kernel(input_index, embedding_weight):
    b, s = input_index.shape
    v, d = embedding_weight.shape
    n = b * s
    idx_flat = input_index.reshape(n).astype(jnp.int32)
    out = _build_gather(n, v, d)(embedding_weight, idx_flat)
    return out.reshape(b, s, d)


# speedup vs baseline: 1.0053x; 1.0053x over previous
import functools

import jax
import jax.numpy as jnp
from jax import lax
from jax.experimental import pallas as pl
from jax.experimental.pallas import tpu as pltpu
from jax.experimental.pallas import tpu_sc as plsc

NUM_WORKERS = 32
BLK = 512
SUB = 16
NSUB = BLK // SUB


@functools.lru_cache(maxsize=None)
def _build_gather(n: int, v: int, d: int):
    b_per_w = n // NUM_WORKERS
    n_blk = b_per_w // BLK
    assert b_per_w * NUM_WORKERS == n and n_blk * BLK == b_per_w
    assert n_blk % 2 == 0

    mesh = plsc.VectorSubcoreMesh(core_axis_name="c", subcore_axis_name="s")

    @functools.partial(
        pl.kernel,
        mesh=mesh,
        out_type=jax.ShapeDtypeStruct((n, d), jnp.float32),
        scratch_types=[
            pltpu.VMEM((b_per_w,), jnp.int32),
            pltpu.VMEM((2 * BLK, d), jnp.float32),
            pltpu.SemaphoreType.DMA,
            pltpu.SemaphoreType.DMA,
            pltpu.SemaphoreType.DMA,
            pltpu.SemaphoreType.DMA,
        ],
        compiler_params=pltpu.CompilerParams(use_tc_tiling_on_sc=False),
    )
    def gather_kernel(table_hbm, idx_hbm, out_hbm, idx_v, rows_v, sem_ga,
                      sem_gb, sem_wa, sem_wb):
        wid = lax.axis_index("s") * 2 + lax.axis_index("c")
        base = wid * b_per_w
        g_sems = (sem_ga, sem_gb)
        w_sems = (sem_wa, sem_wb)

        def fire_gathers(blk, half):
            for j in range(NSUB):
                iv = idx_v[pl.ds(blk * BLK + j * SUB, SUB)]
                pltpu.async_copy(
                    table_hbm.at[iv],
                    rows_v.at[pl.ds(half * BLK + j * SUB, SUB)],
                    g_sems[half],
                )

        def drain_gathers(half):
            pltpu.make_async_copy(
                table_hbm.at[pl.ds(0, BLK)],
                rows_v.at[pl.ds(half * BLK, BLK)],
                g_sems[half],
            ).wait()

        def fire_write(blk, half):
            pltpu.async_copy(
                rows_v.at[pl.ds(half * BLK, BLK)],
                out_hbm.at[pl.ds(base + blk * BLK, BLK)],
                w_sems[half],
            )

        def drain_write(half):
            pltpu.make_async_copy(
                rows_v.at[pl.ds(half * BLK, BLK)],
                out_hbm.at[pl.ds(base, BLK)],
                w_sems[half],
            ).wait()

        pltpu.sync_copy(idx_hbm.at[pl.ds(base, b_per_w)], idx_v)
        fire_gathers(0, 0)
        fire_gathers(1, 1)

        def body(g, carry):
            b = 2 * g
            drain_gathers(0)
            fire_write(b, 0)
            drain_gathers(1)
            fire_write(b + 1, 1)
            drain_write(0)
            fire_gathers(b + 2, 0)
            drain_write(1)
            fire_gathers(b + 3, 1)
            return carry

        lax.fori_loop(0, n_blk // 2 - 1, body, 0)

        drain_gathers(0)
        fire_write(n_blk - 2, 0)
        drain_gathers(1)
        fire_write(n_blk - 1, 1)
        drain_write(0)
        drain_write(1)

    return gather_kernel


def kernel(input_index, embedding_weight):
    b, s = input_index.shape
    v, d = embedding_weight.shape
    n = b * s
    idx_flat = input_index.reshape(n).astype(jnp.int32)
    out = _build_gather(n, v, d)(embedding_weight, idx_flat)
    return out.reshape(b, s, d)


# speedup vs baseline: 1.3370x; 1.3300x over previous
import functools

import jax
import jax.numpy as jnp
from jax import lax
from jax.experimental import pallas as pl
from jax.experimental.pallas import tpu as pltpu
from jax.experimental.pallas import tpu_sc as plsc

NUM_WORKERS = 32
LANES = 16
PITCH = 136


@functools.lru_cache(maxsize=None)
def _build_gather(n_sg: int, d: int):
    seq = n_sg * 8
    n_dg = d // 8
    mesh = plsc.VectorSubcoreMesh(core_axis_name="c", subcore_axis_name="s")

    @functools.partial(
        pl.kernel,
        mesh=mesh,
        out_type=jax.ShapeDtypeStruct((seq, n_dg, NUM_WORKERS, 8, 128),
                                      jnp.float32),
        scratch_types=[
            pltpu.VMEM((n_sg, 8, 128), jnp.int32),
            pltpu.VMEM((2, 128, d), jnp.float32),
            pltpu.VMEM((2, n_dg, 8, PITCH), jnp.float32),
            pltpu.SemaphoreType.DMA,
            pltpu.SemaphoreType.DMA,
            pltpu.SemaphoreType.DMA,
            pltpu.SemaphoreType.DMA,
        ],
        compiler_params=pltpu.CompilerParams(use_tc_tiling_on_sc=False,
                                             needs_layout_passes=False),
    )
    def gather_kernel(table_hbm, idx_hbm, out_hbm, idx_v, rows_v, patch_v,
                      sem_g0, sem_g1, sem_w0, sem_w1):
        wid = lax.axis_index("s") * 2 + lax.axis_index("c")
        g_sems = (sem_g0, sem_g1)
        w_sems = (sem_w0, sem_w1)

        pltpu.sync_copy(idx_hbm.at[:, wid], idx_v)

        def fire_gathers(s, half):
            sg = s // 8
            sr = lax.rem(s, 8)
            for j in range(8):
                iv = idx_v[sg, sr, pl.ds(j * LANES, LANES)]
                pltpu.async_copy(
                    table_hbm.at[iv + iv],
                    rows_v.at[half, pl.ds(j * LANES, LANES)],
                    g_sems[half],
                )

        def drain_gathers(half):
            pltpu.make_async_copy(table_hbm.at[pl.ds(0, 128)],
                                  rows_v.at[half], g_sems[half]).wait()

        def fire_write(s, half):
            pltpu.async_copy(
                patch_v.at[half, :, :, pl.ds(0, 128)],
                out_hbm.at[s, :, wid],
                w_sems[half],
            )

        def drain_write(half):
            pltpu.make_async_copy(patch_v.at[half, :, :, pl.ds(0, 128)],
                                  out_hbm.at[0, :, wid], w_sems[half]).wait()

        lane_iota = lax.iota(jnp.int32, LANES)
        dg_vecs = [(lane_iota >> 3) + (2 * k) for k in range(d // LANES)]
        dr_vec = lane_iota & 7
        zeros = lane_iota - lane_iota

        def transpose_s(half):
            def t_body(i, carry):
                for bi in range(8):
                    b = i * 8 + bi
                    b_vec = zeros + b
                    for k in range(d // LANES):
                        x = rows_v[half, b, pl.ds(k * LANES, LANES)]
                        plsc.store_scatter(patch_v.at[half],
                                           [dg_vecs[k], dr_vec, b_vec], x)
                return carry
            lax.fori_loop(0, 16, t_body, 0)

        fire_gathers(0, 0)

        def body(g, carry):
            s0 = 2 * g
            s1 = s0 + 1
            drain_gathers(0)
            fire_gathers(s1, 1)
            @pl.when(g >= 1)
            def _():
                drain_write(0)
            transpose_s(0)
            fire_write(s0, 0)
            drain_gathers(1)
            @pl.when(g <= (seq // 2) - 2)
            def _():
                fire_gathers(s1 + 1, 0)
            @pl.when(g >= 1)
            def _():
                drain_write(1)
            transpose_s(1)
            fire_write(s1, 1)
            return carry

        lax.fori_loop(0, seq // 2, body, 0)
        drain_write(0)
        drain_write(1)

    return gather_kernel


def kernel(input_index, embedding_weight):
    b, seq = input_index.shape
    v, d = embedding_weight.shape
    n_sg = seq // 8
    tpad = jnp.pad(embedding_weight, ((0, 0), (0, 128 - d)))
    t2 = tpad.reshape(2 * v, d)
    idx4 = (input_index.T.reshape(n_sg, 8, NUM_WORKERS, 128)
            .transpose(0, 2, 1, 3))
    out5 = _build_gather(n_sg, d)(t2, idx4)
    return out5.transpose(2, 4, 0, 1, 3).reshape(b, seq, d)
